# baseline (speedup 1.0000x reference)
import jax
import jax.numpy as jnp
from jax.experimental import pallas as pl
from jax.experimental.pallas import tpu as pltpu

_BATCH = 4
_ROWS = 8192
_DIM = 256
_BLOCK = 1024


def _bcast_body(w_ref, out_ref, sems):
    i = pl.program_id(0)
    copies = [
        pltpu.make_async_copy(
            w_ref,
            out_ref.at[b, pl.ds(i * _BLOCK, _BLOCK), :],
            sems.at[b],
        )
        for b in range(_BATCH)
    ]
    for c in copies:
        c.start()
    for c in copies:
        c.wait()


def kernel(tokens, W):
    del tokens
    grid = (_ROWS // _BLOCK,)
    return pl.pallas_call(
        _bcast_body,
        grid=grid,
        in_specs=[pl.BlockSpec((_BLOCK, _DIM), lambda i: (i, 0))],
        out_specs=pl.BlockSpec(memory_space=pl.ANY),
        out_shape=jax.ShapeDtypeStruct((_BATCH, _ROWS, _DIM), jnp.float32),
        scratch_shapes=[pltpu.SemaphoreType.DMA((_BATCH,))],
        compiler_params=pltpu.CompilerParams(
            dimension_semantics=("arbitrary",),
        ),
    )(W)


# speedup vs baseline: 1.3453x; 1.3453x over previous
import jax
import jax.numpy as jnp
from jax.experimental import pallas as pl

_BATCH = 4
_ROWS = 8192
_DIM = 256
_BLOCK = 2048


def _bcast_body(w_ref, out_ref):
    out_ref[...] = jnp.broadcast_to(w_ref[...][None], (_BATCH, _BLOCK, _DIM))


def kernel(tokens, W):
    del tokens
    grid = (_ROWS // _BLOCK,)
    return pl.pallas_call(
        _bcast_body,
        grid=grid,
        in_specs=[pl.BlockSpec((_BLOCK, _DIM), lambda i: (i, 0))],
        out_specs=pl.BlockSpec((_BATCH, _BLOCK, _DIM), lambda i: (0, i, 0)),
        out_shape=jax.ShapeDtypeStruct((_BATCH, _ROWS, _DIM), jnp.float32),
    )(W)
